# Initial kernel scaffold; baseline (speedup 1.0000x reference)
import jax
import jax.numpy as jnp
from jax.experimental import pallas as pl


def kernel(x, edge_index, edge_attr, Wl, bl, W1, b1, W2, b2, gamma, beta):
    raise NotImplementedError("write your pallas kernel here")


# speedup vs baseline: 3.1258x; 3.1258x over previous
import functools

import jax
import jax.numpy as jnp
from jax import lax
from jax.experimental import pallas as pl
from jax.experimental.pallas import tpu as pltpu
from jax.experimental.pallas import tpu_sc as plsc

N = 10000
E = 320000
D = 128
EPS_BN = 1e-5

ROW = 128
NROWS = E // ROW
NSC = 2
NSUB = 16
NW = NSC * NSUB
N_PAD = 10240
SLAB = N_PAD // NSUB


def _edge_linear_body(a_ref, w_ref, b_ref, o_ref):
    o_ref[...] = (
        jnp.dot(a_ref[...], w_ref[...], preferred_element_type=jnp.float32)
        + b_ref[...]
    )


def _edge_linear(edge_attr, wlt, bl2):
    B = 2560
    return pl.pallas_call(
        _edge_linear_body,
        grid=(E // B,),
        in_specs=[
            pl.BlockSpec((B, D), lambda i: (i, 0)),
            pl.BlockSpec((D, D), lambda i: (0, 0)),
            pl.BlockSpec((1, D), lambda i: (0, 0)),
        ],
        out_specs=pl.BlockSpec((B, D), lambda i: (i, 0)),
        out_shape=jax.ShapeDtypeStruct((E, D), jnp.float32),
    )(edge_attr, wlt, bl2)


def _sc_aggregate(P, x, src2, dst2, zeros):
    mesh = plsc.VectorSubcoreMesh(core_axis_name="c", subcore_axis_name="s")

    @functools.partial(
        pl.kernel,
        mesh=mesh,
        out_type=jax.ShapeDtypeStruct((NSC, N_PAD, D), jnp.float32),
        scratch_types=[
            pltpu.VMEM((ROW,), jnp.int32),
            pltpu.VMEM((ROW,), jnp.int32),
            pltpu.VMEM((ROW, D), jnp.float32),
            pltpu.VMEM((ROW, D), jnp.float32),
            pltpu.VMEM_SHARED((N_PAD, D), jnp.float32),
            pltpu.SemaphoreType.DMA,
        ],
    )
    def body(p_hbm, x_hbm, src_hbm, dst_hbm, z_hbm, out_hbm,
             src_v, dst_v, xg_v, m_v, aggr_sh, sem):
        c = lax.axis_index("c")
        s = lax.axis_index("s")
        w = c * NSUB + s

        pltpu.sync_copy(z_hbm, aggr_sh.at[pl.ds(s * SLAB, SLAB)])
        plsc.subcore_barrier()

        lo = (w * NROWS) // NW
        hi = ((w + 1) * NROWS) // NW

        def chunk(r, carry):
            pltpu.sync_copy(src_hbm.at[r], src_v)
            pltpu.sync_copy(dst_hbm.at[r], dst_v)
            pltpu.async_copy(x_hbm.at[src_v], xg_v, sem).wait()
            pltpu.sync_copy(p_hbm.at[pl.ds(r * ROW, ROW)], m_v)

            def rowfn(i, carry2):
                for j in range(D // 16):
                    sl = pl.ds(j * 16, 16)
                    m_v[i, sl] = jnp.maximum(xg_v[i, sl] + m_v[i, sl], 0.0)
                return carry2

            lax.fori_loop(0, ROW, rowfn, 0)
            pltpu.sync_copy(m_v, aggr_sh.at[dst_v], add=True)
            return carry

        lax.fori_loop(lo, hi, chunk, 0)
        plsc.subcore_barrier()
        pltpu.sync_copy(aggr_sh.at[pl.ds(s * SLAB, SLAB)],
                        out_hbm.at[c, pl.ds(s * SLAB, SLAB)])

    return body(P, x, src2, dst2, zeros)


def _finalize_body(x_ref, p_ref, w1_ref, b1_ref, w2_ref, b2_ref,
                   g_ref, be_ref, o_ref):
    h0 = x_ref[...] + p_ref[0] + p_ref[1]
    h1 = jnp.maximum(
        jnp.dot(h0, w1_ref[...], preferred_element_type=jnp.float32)
        + b1_ref[...], 0.0)
    h = (jnp.dot(h1, w2_ref[...], preferred_element_type=jnp.float32)
         + b2_ref[...])
    mean = jnp.mean(h, axis=0, keepdims=True)
    cent = h - mean
    var = jnp.mean(cent * cent, axis=0, keepdims=True)
    o_ref[...] = cent * lax.rsqrt(var + EPS_BN) * g_ref[...] + be_ref[...]


def _finalize(x, parts, w1t, b12, w2t, b22, g2, be2):
    return pl.pallas_call(
        _finalize_body,
        out_shape=jax.ShapeDtypeStruct((N, D), jnp.float32),
    )(x, parts, w1t, b12, w2t, b22, g2, be2)


def kernel(x, edge_index, edge_attr, Wl, bl, W1, b1, W2, b2, gamma, beta):
    src2 = edge_index[0].reshape(NROWS, ROW)
    dst2 = edge_index[1].reshape(NROWS, ROW)
    P = _edge_linear(edge_attr, Wl.T, bl.reshape(1, D))
    zeros = jnp.zeros((SLAB, D), jnp.float32)
    parts = _sc_aggregate(P, x, src2, dst2, zeros)[:, :N]
    return _finalize(x, parts, W1.T, b1.reshape(1, D), W2.T, b2.reshape(1, D),
                     gamma.reshape(1, D), beta.reshape(1, D))


# speedup vs baseline: 4.5122x; 1.4435x over previous
import functools

import jax
import jax.numpy as jnp
from jax import lax
from jax.experimental import pallas as pl
from jax.experimental.pallas import tpu as pltpu
from jax.experimental.pallas import tpu_sc as plsc

N = 10000
E = 320000
D = 128
EPS_BN = 1e-5

ROW = 64
NROWS = E // ROW
NSC = 2
NSUB = 16
NW = NSC * NSUB
N_PAD = 10112
SLAB = N_PAD // NSUB
NROWS_PAD = 5120
IDX_ROWS = NROWS_PAD + 2
CHUNKS = NROWS_PAD // NW


def _edge_linear_body(a_ref, w_ref, b_ref, o_ref):
    o_ref[...] = (
        jnp.dot(a_ref[...], w_ref[...], preferred_element_type=jnp.float32)
        + b_ref[...]
    )


def _edge_linear(edge_attr, wlt, bl2):
    B = 2560
    return pl.pallas_call(
        _edge_linear_body,
        grid=(E // B,),
        in_specs=[
            pl.BlockSpec((B, D), lambda i: (i, 0)),
            pl.BlockSpec((D, D), lambda i: (0, 0)),
            pl.BlockSpec((1, D), lambda i: (0, 0)),
        ],
        out_specs=pl.BlockSpec((B, D), lambda i: (i, 0)),
        out_shape=jax.ShapeDtypeStruct((E, D), jnp.float32),
    )(edge_attr, wlt, bl2)


def _sc_aggregate(P, x, src2, dst2, zeros):
    mesh = plsc.VectorSubcoreMesh(core_axis_name="c", subcore_axis_name="s")

    @functools.partial(
        pl.kernel,
        mesh=mesh,
        out_type=jax.ShapeDtypeStruct((NSC, N_PAD, D), jnp.float32),
        scratch_types=[
            pltpu.VMEM((4, ROW), jnp.int32),
            pltpu.VMEM((4, ROW), jnp.int32),
            pltpu.VMEM((2, ROW, D), jnp.float32),
            pltpu.VMEM((2, ROW, D), jnp.float32),
            pltpu.VMEM((2, ROW, D), jnp.float32),
            pltpu.VMEM_SHARED((N_PAD, D), jnp.float32),
            pltpu.SemaphoreType.DMA,
            pltpu.SemaphoreType.DMA,
            pltpu.SemaphoreType.DMA,
            pltpu.SemaphoreType.DMA,
        ],
    )
    def body(p_hbm, x_hbm, src_hbm, dst_hbm, z_hbm, out_hbm,
             src_i, dst_i, xg_v, pm_v, mb_v, aggr_sh,
             s_f, s_g, s_s0, s_s1):
        c = lax.axis_index("c")
        s = lax.axis_index("s")
        w = c * NSUB + s
        s_s = (s_s0, s_s1)

        pltpu.sync_copy(z_hbm, aggr_sh.at[pl.ds(s * SLAB, SLAB)])
        plsc.subcore_barrier()

        c0 = w * CHUNKS

        def f_descs(r):
            slot = lax.rem(r, 4)
            return (pltpu.make_async_copy(src_hbm.at[r], src_i.at[slot], s_f),
                    pltpu.make_async_copy(dst_hbm.at[r], dst_i.at[slot], s_f))

        def g_descs(r, b):
            slot = lax.rem(r, 4)
            rp = jnp.minimum(r, NROWS - 1)
            return (
                pltpu.make_async_copy(x_hbm.at[src_i.at[slot]], xg_v.at[b],
                                      s_g),
                pltpu.make_async_copy(p_hbm.at[pl.ds(rp * ROW, ROW)],
                                      pm_v.at[b], s_g),
            )

        def s_desc(r, b):
            slot = lax.rem(r, 4)
            return pltpu.make_async_copy(mb_v.at[b], aggr_sh.at[dst_i.at[slot]],
                                         s_s[b])

        def start(descs):
            for d in descs:
                d.start()

        def wait(descs):
            for d in descs:
                d.wait()

        def compute(b):
            xgb, pmb, mbb = xg_v.at[b], pm_v.at[b], mb_v.at[b]

            @plsc.parallel_loop(0, ROW, unroll=2)
            def _(i):
                for j in range(D // 16):
                    sl = pl.ds(j * 16, 16)
                    mbb[i, sl] = jnp.maximum(xgb[i, sl] + pmb[i, sl], 0.0)

        start(f_descs(c0))
        wait(f_descs(c0))
        start(g_descs(c0, 0))
        start(f_descs(c0 + 1))

        def step(t, carry):
            r0 = c0 + 2 * t
            r1 = r0 + 1
            wait(g_descs(r0, 0))
            wait(f_descs(r1))
            start(g_descs(r1, 1))
            start(f_descs(r0 + 2))

            @pl.when(t > 0)
            def _():
                s_desc(r0 - 2, 0).wait()

            compute(0)
            pltpu.async_copy(mb_v.at[0], aggr_sh.at[dst_i.at[lax.rem(r0, 4)]],
                             s_s[0], add=True)
            wait(g_descs(r1, 1))
            wait(f_descs(r0 + 2))
            start(g_descs(r0 + 2, 0))
            start(f_descs(r1 + 2))

            @pl.when(t > 0)
            def _():
                s_desc(r1 - 2, 1).wait()

            compute(1)
            pltpu.async_copy(mb_v.at[1], aggr_sh.at[dst_i.at[lax.rem(r1, 4)]],
                             s_s[1], add=True)
            return carry

        lax.fori_loop(0, CHUNKS // 2, step, 0)

        s_desc(c0 + CHUNKS - 2, 0).wait()
        s_desc(c0 + CHUNKS - 1, 1).wait()
        wait(g_descs(c0 + CHUNKS, 0))
        wait(f_descs(c0 + CHUNKS + 1))

        plsc.subcore_barrier()
        pltpu.sync_copy(aggr_sh.at[pl.ds(s * SLAB, SLAB)],
                        out_hbm.at[c, pl.ds(s * SLAB, SLAB)])

    return body(P, x, src2, dst2, zeros)


def _finalize_body(x_ref, p_ref, w1_ref, b1_ref, w2_ref, b2_ref,
                   g_ref, be_ref, o_ref):
    h0 = x_ref[...] + p_ref[0] + p_ref[1]
    h1 = jnp.maximum(
        jnp.dot(h0, w1_ref[...], preferred_element_type=jnp.float32)
        + b1_ref[...], 0.0)
    h = (jnp.dot(h1, w2_ref[...], preferred_element_type=jnp.float32)
         + b2_ref[...])
    mean = jnp.mean(h, axis=0, keepdims=True)
    cent = h - mean
    var = jnp.mean(cent * cent, axis=0, keepdims=True)
    o_ref[...] = cent * lax.rsqrt(var + EPS_BN) * g_ref[...] + be_ref[...]


def _finalize(x, parts, w1t, b12, w2t, b22, g2, be2):
    return pl.pallas_call(
        _finalize_body,
        out_shape=jax.ShapeDtypeStruct((N, D), jnp.float32),
    )(x, parts, w1t, b12, w2t, b22, g2, be2)


def kernel(x, edge_index, edge_attr, Wl, bl, W1, b1, W2, b2, gamma, beta):
    pad_e = (IDX_ROWS - NROWS) * ROW
    ar = jnp.arange(pad_e, dtype=jnp.int32)
    src2 = jnp.concatenate([edge_index[0], ar % N]).reshape(IDX_ROWS, ROW)
    dst2 = jnp.concatenate([edge_index[1], N + ar % (N_PAD - N)]
                           ).reshape(IDX_ROWS, ROW)
    P = _edge_linear(edge_attr, Wl.T, bl.reshape(1, D))
    zeros = jnp.zeros((SLAB, D), jnp.float32)
    parts = _sc_aggregate(P, x, src2, dst2, zeros)[:, :N]
    return _finalize(x, parts, W1.T, b1.reshape(1, D), W2.T, b2.reshape(1, D),
                     gamma.reshape(1, D), beta.reshape(1, D))


# speedup vs baseline: 4.5147x; 1.0006x over previous
import functools

import jax
import jax.numpy as jnp
from jax import lax
from jax.experimental import pallas as pl
from jax.experimental.pallas import tpu as pltpu
from jax.experimental.pallas import tpu_sc as plsc

N = 10000
E = 320000
D = 128
EPS_BN = 1e-5

ROW = 64
NROWS = E // ROW
NSC = 2
NSUB = 16
NW = NSC * NSUB
N_PAD = 10112
SLAB = N_PAD // NSUB
NROWS_PAD = 5120
IDX_ROWS = NROWS_PAD + 2
CHUNKS = NROWS_PAD // NW


def _edge_linear_body(a_ref, w_ref, b_ref, o_ref):
    o_ref[...] = (
        jnp.dot(a_ref[...], w_ref[...], preferred_element_type=jnp.float32)
        + b_ref[...]
    )


def _edge_linear(edge_attr, wlt, bl2):
    B = 2560
    return pl.pallas_call(
        _edge_linear_body,
        grid=(E // B,),
        in_specs=[
            pl.BlockSpec((B, D), lambda i: (i, 0)),
            pl.BlockSpec((D, D), lambda i: (0, 0)),
            pl.BlockSpec((1, D), lambda i: (0, 0)),
        ],
        out_specs=pl.BlockSpec((B, D), lambda i: (i, 0)),
        out_shape=jax.ShapeDtypeStruct((E, D), jnp.float32),
    )(edge_attr, wlt, bl2)


def _sc_aggregate(P, x, src2, dst2, zeros):
    mesh = plsc.VectorSubcoreMesh(core_axis_name="c", subcore_axis_name="s")

    @functools.partial(
        pl.kernel,
        mesh=mesh,
        out_type=jax.ShapeDtypeStruct((NSC, N_PAD, D), jnp.float32),
        scratch_types=[
            pltpu.VMEM((4, ROW), jnp.int32),
            pltpu.VMEM((4, ROW), jnp.int32),
            pltpu.VMEM((2, ROW, D), jnp.float32),
            pltpu.VMEM((ROW, D), jnp.float32),
            pltpu.VMEM((ROW, D), jnp.float32),
            pltpu.VMEM((2, ROW, D), jnp.float32),
            pltpu.VMEM_SHARED((N_PAD, D), jnp.float32),
            pltpu.SemaphoreType.DMA,
            pltpu.SemaphoreType.DMA,
            pltpu.SemaphoreType.DMA,
            pltpu.SemaphoreType.DMA,
        ],
    )
    def body(p_hbm, x_hbm, src_hbm, dst_hbm, z_hbm, out_hbm,
             src_i, dst_i, xg_v, pm0_v, pm1_v, mb_v, aggr_sh,
             s_f, s_g, s_s0, s_s1):
        pm_v = (pm0_v, pm1_v)
        c = lax.axis_index("c")
        s = lax.axis_index("s")
        w = c * NSUB + s
        s_s = (s_s0, s_s1)

        pltpu.sync_copy(z_hbm, aggr_sh.at[pl.ds(s * SLAB, SLAB)])
        plsc.subcore_barrier()

        c0 = w * CHUNKS

        def f_descs(r):
            slot = lax.rem(r, 4)
            return (pltpu.make_async_copy(src_hbm.at[r], src_i.at[slot], s_f),
                    pltpu.make_async_copy(dst_hbm.at[r], dst_i.at[slot], s_f))

        def g_descs(r, b):
            slot = lax.rem(r, 4)
            rp = jnp.minimum(r, NROWS - 1)
            return (
                pltpu.make_async_copy(x_hbm.at[src_i.at[slot]], xg_v.at[b],
                                      s_g),
                pltpu.make_async_copy(p_hbm.at[pl.ds(rp * ROW, ROW)],
                                      pm_v[b], s_g),
            )

        def s_desc(r, b):
            slot = lax.rem(r, 4)
            return pltpu.make_async_copy(mb_v.at[b], aggr_sh.at[dst_i.at[slot]],
                                         s_s[b])

        def start(descs):
            for d in descs:
                d.start()

        def wait(descs):
            for d in descs:
                d.wait()

        def compute(b):
            xgb, pmb, mbb = xg_v.at[b], pm_v[b], mb_v.at[b]

            @plsc.parallel_loop(0, ROW, unroll=4)
            def _(i):
                for j in range(D // 16):
                    sl = pl.ds(j * 16, 16)
                    mbb[i, sl] = jnp.maximum(xgb[i, sl] + pmb[i, sl], 0.0)

        start(f_descs(c0))
        wait(f_descs(c0))
        start(g_descs(c0, 0))
        start(f_descs(c0 + 1))

        def step(t, carry):
            r0 = c0 + 2 * t
            r1 = r0 + 1
            wait(g_descs(r0, 0))
            wait(f_descs(r1))
            start(g_descs(r1, 1))
            start(f_descs(r0 + 2))

            @pl.when(t > 0)
            def _():
                s_desc(r0 - 2, 0).wait()

            compute(0)
            pltpu.async_copy(mb_v.at[0], aggr_sh.at[dst_i.at[lax.rem(r0, 4)]],
                             s_s[0], add=True)
            wait(g_descs(r1, 1))
            wait(f_descs(r0 + 2))
            start(g_descs(r0 + 2, 0))
            start(f_descs(r1 + 2))

            @pl.when(t > 0)
            def _():
                s_desc(r1 - 2, 1).wait()

            compute(1)
            pltpu.async_copy(mb_v.at[1], aggr_sh.at[dst_i.at[lax.rem(r1, 4)]],
                             s_s[1], add=True)
            return carry

        lax.fori_loop(0, CHUNKS // 2, step, 0)

        s_desc(c0 + CHUNKS - 2, 0).wait()
        s_desc(c0 + CHUNKS - 1, 1).wait()
        wait(g_descs(c0 + CHUNKS, 0))
        wait(f_descs(c0 + CHUNKS + 1))

        plsc.subcore_barrier()
        pltpu.sync_copy(aggr_sh.at[pl.ds(s * SLAB, SLAB)],
                        out_hbm.at[c, pl.ds(s * SLAB, SLAB)])

    return body(P, x, src2, dst2, zeros)


def _finalize_body(x_ref, p_ref, w1_ref, b1_ref, w2_ref, b2_ref,
                   g_ref, be_ref, o_ref):
    h0 = x_ref[...] + p_ref[0] + p_ref[1]
    h1 = jnp.maximum(
        jnp.dot(h0, w1_ref[...], preferred_element_type=jnp.float32)
        + b1_ref[...], 0.0)
    h = (jnp.dot(h1, w2_ref[...], preferred_element_type=jnp.float32)
         + b2_ref[...])
    mean = jnp.mean(h, axis=0, keepdims=True)
    cent = h - mean
    var = jnp.mean(cent * cent, axis=0, keepdims=True)
    o_ref[...] = cent * lax.rsqrt(var + EPS_BN) * g_ref[...] + be_ref[...]


def _finalize(x, parts, w1t, b12, w2t, b22, g2, be2):
    return pl.pallas_call(
        _finalize_body,
        out_shape=jax.ShapeDtypeStruct((N, D), jnp.float32),
    )(x, parts, w1t, b12, w2t, b22, g2, be2)


def kernel(x, edge_index, edge_attr, Wl, bl, W1, b1, W2, b2, gamma, beta):
    pad_e = (IDX_ROWS - NROWS) * ROW
    ar = jnp.arange(pad_e, dtype=jnp.int32)
    src2 = jnp.concatenate([edge_index[0], ar % N]).reshape(IDX_ROWS, ROW)
    dst2 = jnp.concatenate([edge_index[1], N + ar % (N_PAD - N)]
                           ).reshape(IDX_ROWS, ROW)
    P = _edge_linear(edge_attr, Wl.T, bl.reshape(1, D))
    zeros = jnp.zeros((SLAB, D), jnp.float32)
    parts = _sc_aggregate(P, x, src2, dst2, zeros)[:, :N]
    return _finalize(x, parts, W1.T, b1.reshape(1, D), W2.T, b2.reshape(1, D),
                     gamma.reshape(1, D), beta.reshape(1, D))


# speedup vs baseline: 4.6125x; 1.0217x over previous
import functools

import jax
import jax.numpy as jnp
from jax import lax
from jax.experimental import pallas as pl
from jax.experimental.pallas import tpu as pltpu
from jax.experimental.pallas import tpu_sc as plsc

N = 10000
E = 320000
D = 128
EPS_BN = 1e-5

ROW = 64
NROWS = E // ROW
NSC = 2
NSUB = 16
NW = NSC * NSUB
N_PAD = 10112
SLAB = N_PAD // NSUB
E_HALF = E // 2
NROWS_H = E_HALF // ROW
NROWS_PAD_H = 2560
IDX_ROWS_H = NROWS_PAD_H + 2
CHUNKS = NROWS_PAD_H // NW


def _edge_linear_body(a_ref, w_ref, b_ref, o_ref):
    o_ref[...] = (
        jnp.dot(a_ref[...], w_ref[...], preferred_element_type=jnp.float32)
        + b_ref[...]
    )


def _edge_linear(edge_attr, wlt, bl2, base_blk):
    B = 2000
    return pl.pallas_call(
        _edge_linear_body,
        grid=(E_HALF // B,),
        in_specs=[
            pl.BlockSpec((B, D), lambda i: (i + base_blk, 0)),
            pl.BlockSpec((D, D), lambda i: (0, 0)),
            pl.BlockSpec((1, D), lambda i: (0, 0)),
        ],
        out_specs=pl.BlockSpec((B, D), lambda i: (i, 0)),
        out_shape=jax.ShapeDtypeStruct((E_HALF, D), jnp.float32),
    )(edge_attr, wlt, bl2)


def _sc_aggregate(P, x, src2, dst2, zeros):
    mesh = plsc.VectorSubcoreMesh(core_axis_name="c", subcore_axis_name="s")

    @functools.partial(
        pl.kernel,
        mesh=mesh,
        out_type=jax.ShapeDtypeStruct((NSC, N_PAD, D), jnp.float32),
        scratch_types=[
            pltpu.VMEM((4, ROW), jnp.int32),
            pltpu.VMEM((4, ROW), jnp.int32),
            pltpu.VMEM((2, ROW, D), jnp.float32),
            pltpu.VMEM((ROW, D), jnp.float32),
            pltpu.VMEM((ROW, D), jnp.float32),
            pltpu.VMEM((2, ROW, D), jnp.float32),
            pltpu.VMEM_SHARED((N_PAD, D), jnp.float32),
            pltpu.SemaphoreType.DMA,
            pltpu.SemaphoreType.DMA,
            pltpu.SemaphoreType.DMA,
            pltpu.SemaphoreType.DMA,
        ],
    )
    def body(p_hbm, x_hbm, src_hbm, dst_hbm, z_hbm, out_hbm,
             src_i, dst_i, xg_v, pm0_v, pm1_v, mb_v, aggr_sh,
             s_f, s_g, s_s0, s_s1):
        pm_v = (pm0_v, pm1_v)
        c = lax.axis_index("c")
        s = lax.axis_index("s")
        w = c * NSUB + s
        s_s = (s_s0, s_s1)

        pltpu.sync_copy(z_hbm, aggr_sh.at[pl.ds(s * SLAB, SLAB)])
        plsc.subcore_barrier()

        c0 = w * CHUNKS

        def f_descs(r):
            slot = lax.rem(r, 4)
            return (pltpu.make_async_copy(src_hbm.at[r], src_i.at[slot], s_f),
                    pltpu.make_async_copy(dst_hbm.at[r], dst_i.at[slot], s_f))

        def g_descs(r, b):
            slot = lax.rem(r, 4)
            rp = jnp.minimum(r, NROWS_H - 1)
            return (
                pltpu.make_async_copy(x_hbm.at[src_i.at[slot]], xg_v.at[b],
                                      s_g),
                pltpu.make_async_copy(p_hbm.at[pl.ds(rp * ROW, ROW)],
                                      pm_v[b], s_g),
            )

        def s_desc(r, b):
            slot = lax.rem(r, 4)
            return pltpu.make_async_copy(mb_v.at[b], aggr_sh.at[dst_i.at[slot]],
                                         s_s[b])

        def start(descs):
            for d in descs:
                d.start()

        def wait(descs):
            for d in descs:
                d.wait()

        def compute(b):
            xgb, pmb, mbb = xg_v.at[b], pm_v[b], mb_v.at[b]

            @plsc.parallel_loop(0, ROW, unroll=4)
            def _(i):
                for j in range(D // 16):
                    sl = pl.ds(j * 16, 16)
                    mbb[i, sl] = jnp.maximum(xgb[i, sl] + pmb[i, sl], 0.0)

        start(f_descs(c0))
        wait(f_descs(c0))
        start(g_descs(c0, 0))
        start(f_descs(c0 + 1))

        def step(t, carry):
            r0 = c0 + 2 * t
            r1 = r0 + 1
            wait(g_descs(r0, 0))
            wait(f_descs(r1))
            start(g_descs(r1, 1))
            start(f_descs(r0 + 2))

            @pl.when(t > 0)
            def _():
                s_desc(r0 - 2, 0).wait()

            compute(0)
            pltpu.async_copy(mb_v.at[0], aggr_sh.at[dst_i.at[lax.rem(r0, 4)]],
                             s_s[0], add=True)
            wait(g_descs(r1, 1))
            wait(f_descs(r0 + 2))
            start(g_descs(r0 + 2, 0))
            start(f_descs(r1 + 2))

            @pl.when(t > 0)
            def _():
                s_desc(r1 - 2, 1).wait()

            compute(1)
            pltpu.async_copy(mb_v.at[1], aggr_sh.at[dst_i.at[lax.rem(r1, 4)]],
                             s_s[1], add=True)
            return carry

        lax.fori_loop(0, CHUNKS // 2, step, 0)

        s_desc(c0 + CHUNKS - 2, 0).wait()
        s_desc(c0 + CHUNKS - 1, 1).wait()
        wait(g_descs(c0 + CHUNKS, 0))
        wait(f_descs(c0 + CHUNKS + 1))

        plsc.subcore_barrier()
        pltpu.sync_copy(aggr_sh.at[pl.ds(s * SLAB, SLAB)],
                        out_hbm.at[c, pl.ds(s * SLAB, SLAB)])

    return body(P, x, src2, dst2, zeros)


def _finalize_body(x_ref, pa_ref, pb_ref, w1_ref, b1_ref, w2_ref, b2_ref,
                   g_ref, be_ref, o_ref):
    h0 = (x_ref[...] + pa_ref[0] + pa_ref[1]) + (pb_ref[0] + pb_ref[1])
    h1 = jnp.maximum(
        jnp.dot(h0, w1_ref[...], preferred_element_type=jnp.float32)
        + b1_ref[...], 0.0)
    h = (jnp.dot(h1, w2_ref[...], preferred_element_type=jnp.float32)
         + b2_ref[...])
    mean = jnp.mean(h, axis=0, keepdims=True)
    cent = h - mean
    var = jnp.mean(cent * cent, axis=0, keepdims=True)
    o_ref[...] = cent * lax.rsqrt(var + EPS_BN) * g_ref[...] + be_ref[...]


def _finalize(x, parts_a, parts_b, w1t, b12, w2t, b22, g2, be2):
    return pl.pallas_call(
        _finalize_body,
        out_shape=jax.ShapeDtypeStruct((N, D), jnp.float32),
    )(x, parts_a, parts_b, w1t, b12, w2t, b22, g2, be2)


def kernel(x, edge_index, edge_attr, Wl, bl, W1, b1, W2, b2, gamma, beta):
    pad_e = IDX_ROWS_H * ROW - E_HALF
    ar = jnp.arange(pad_e, dtype=jnp.int32)
    src_pad = ar % N
    dst_pad = N + ar % (N_PAD - N)
    src = edge_index[0]
    dst = edge_index[1]
    idx = []
    for h in range(2):
        sl = slice(h * E_HALF, (h + 1) * E_HALF)
        idx.append((
            jnp.concatenate([src[sl], src_pad]).reshape(IDX_ROWS_H, ROW),
            jnp.concatenate([dst[sl], dst_pad]).reshape(IDX_ROWS_H, ROW),
        ))
    wlt = Wl.T
    bl2 = bl.reshape(1, D)
    zeros = jnp.zeros((SLAB, D), jnp.float32)
    nblk = E_HALF // 2000
    p0 = _edge_linear(edge_attr, wlt, bl2, 0)
    p1 = _edge_linear(edge_attr, wlt, bl2, nblk)
    parts_a = _sc_aggregate(p0, x, idx[0][0], idx[0][1], zeros)[:, :N]
    parts_b = _sc_aggregate(p1, x, idx[1][0], idx[1][1], zeros)[:, :N]
    return _finalize(x, parts_a, parts_b, W1.T, b1.reshape(1, D), W2.T,
                     b2.reshape(1, D), gamma.reshape(1, D), beta.reshape(1, D))


# speedup vs baseline: 4.6681x; 1.0121x over previous
import functools

import jax
import jax.numpy as jnp
from jax import lax
from jax.experimental import pallas as pl
from jax.experimental.pallas import tpu as pltpu
from jax.experimental.pallas import tpu_sc as plsc

N = 10000
E = 320000
D = 128
EPS_BN = 1e-5

ROW = 64
NROWS = E // ROW
NSC = 2
NSUB = 16
NW = NSC * NSUB
N_PAD = 10112
SLAB = N_PAD // NSUB
E_HALF = E // 2
NROWS_H = E_HALF // ROW
NROWS_PAD_H = 2560
IDX_ROWS_H = NROWS_PAD_H + 2
CHUNKS = NROWS_PAD_H // NW


def _edge_linear_body(a_ref, w_ref, b_ref, o_ref):
    o_ref[...] = (
        jnp.dot(a_ref[...], w_ref[...], preferred_element_type=jnp.float32)
        + b_ref[...]
    )


def _edge_linear(edge_attr, wlt, bl2, base_blk):
    B = 2000
    return pl.pallas_call(
        _edge_linear_body,
        grid=(E_HALF // B,),
        in_specs=[
            pl.BlockSpec((B, D), lambda i: (i + base_blk, 0)),
            pl.BlockSpec((D, D), lambda i: (0, 0)),
            pl.BlockSpec((1, D), lambda i: (0, 0)),
        ],
        out_specs=pl.BlockSpec((B, D), lambda i: (i, 0)),
        out_shape=jax.ShapeDtypeStruct((E_HALF, D), jnp.float32),
    )(edge_attr, wlt, bl2)


def _sc_aggregate(P, x, src2, dst2, zeros):
    mesh = plsc.VectorSubcoreMesh(core_axis_name="c", subcore_axis_name="s")

    @functools.partial(
        pl.kernel,
        mesh=mesh,
        out_type=jax.ShapeDtypeStruct((NSC, N_PAD, D), jnp.float32),
        scratch_types=[
            pltpu.VMEM((4, ROW), jnp.int32),
            pltpu.VMEM((4, ROW), jnp.int32),
            pltpu.VMEM((2, ROW, D), jnp.float32),
            pltpu.VMEM((ROW, D), jnp.float32),
            pltpu.VMEM((ROW, D), jnp.float32),
            pltpu.VMEM((2, ROW, D), jnp.float32),
            pltpu.VMEM_SHARED((N_PAD, D), jnp.float32),
            pltpu.SemaphoreType.DMA,
            pltpu.SemaphoreType.DMA,
            pltpu.SemaphoreType.DMA,
            pltpu.SemaphoreType.DMA,
        ],
    )
    def body(p_hbm, x_hbm, src_hbm, dst_hbm, z_hbm, out_hbm,
             src_i, dst_i, xg_v, pm0_v, pm1_v, mb_v, aggr_sh,
             s_f, s_g, s_s0, s_s1):
        pm_v = (pm0_v, pm1_v)
        c = lax.axis_index("c")
        s = lax.axis_index("s")
        w = c * NSUB + s
        s_s = (s_s0, s_s1)

        pltpu.sync_copy(z_hbm, aggr_sh.at[pl.ds(s * SLAB, SLAB)])
        plsc.subcore_barrier()

        c0 = w * CHUNKS

        def f_descs(r):
            slot = lax.rem(r, 4)
            return (pltpu.make_async_copy(src_hbm.at[r], src_i.at[slot], s_f),
                    pltpu.make_async_copy(dst_hbm.at[r], dst_i.at[slot], s_f))

        def g_descs(r, b):
            slot = lax.rem(r, 4)
            rp = jnp.minimum(r, NROWS_H - 1)
            return (
                pltpu.make_async_copy(x_hbm.at[src_i.at[slot]], xg_v.at[b],
                                      s_g),
                pltpu.make_async_copy(p_hbm.at[pl.ds(rp * ROW, ROW)],
                                      pm_v[b], s_g),
            )

        def s_desc(r, b):
            slot = lax.rem(r, 4)
            return pltpu.make_async_copy(mb_v.at[b], aggr_sh.at[dst_i.at[slot]],
                                         s_s[b])

        def start(descs):
            for d in descs:
                d.start()

        def wait(descs):
            for d in descs:
                d.wait()

        def compute(b):
            xgb, pmb, mbb = xg_v.at[b], pm_v[b], mb_v.at[b]

            @plsc.parallel_loop(0, ROW, unroll=4)
            def _(i):
                for j in range(D // 16):
                    sl = pl.ds(j * 16, 16)
                    mbb[i, sl] = jnp.maximum(xgb[i, sl] + pmb[i, sl], 0.0)

        start(f_descs(c0))
        wait(f_descs(c0))
        start(g_descs(c0, 0))
        start(f_descs(c0 + 1))

        def step(t, carry):
            r0 = c0 + 2 * t
            r1 = r0 + 1
            wait(g_descs(r0, 0))
            wait(f_descs(r1))
            start(g_descs(r1, 1))
            start(f_descs(r0 + 2))

            @pl.when(t > 0)
            def _():
                s_desc(r0 - 2, 0).wait()

            compute(0)
            pltpu.async_copy(mb_v.at[0], aggr_sh.at[dst_i.at[lax.rem(r0, 4)]],
                             s_s[0], add=True)
            wait(g_descs(r1, 1))
            wait(f_descs(r0 + 2))
            start(g_descs(r0 + 2, 0))
            start(f_descs(r1 + 2))

            @pl.when(t > 0)
            def _():
                s_desc(r1 - 2, 1).wait()

            compute(1)
            pltpu.async_copy(mb_v.at[1], aggr_sh.at[dst_i.at[lax.rem(r1, 4)]],
                             s_s[1], add=True)
            return carry

        lax.fori_loop(0, CHUNKS // 2, step, 0)

        s_desc(c0 + CHUNKS - 2, 0).wait()
        s_desc(c0 + CHUNKS - 1, 1).wait()
        wait(g_descs(c0 + CHUNKS, 0))
        wait(f_descs(c0 + CHUNKS + 1))

        plsc.subcore_barrier()
        pltpu.sync_copy(aggr_sh.at[pl.ds(s * SLAB, SLAB)],
                        out_hbm.at[c, pl.ds(s * SLAB, SLAB)])

    return body(P, x, src2, dst2, zeros)


def _finalize_body(x_ref, pa_ref, pb_ref, w1_ref, b1_ref, w2_ref, b2_ref,
                   g_ref, be_ref, o_ref):
    sl = pl.ds(0, N)
    h0 = ((x_ref[...] + pa_ref[0, sl] + pa_ref[1, sl])
          + (pb_ref[0, sl] + pb_ref[1, sl]))
    h1 = jnp.maximum(
        jnp.dot(h0, w1_ref[...], preferred_element_type=jnp.float32)
        + b1_ref[...], 0.0)
    h = (jnp.dot(h1, w2_ref[...], preferred_element_type=jnp.float32)
         + b2_ref[...])
    mean = jnp.mean(h, axis=0, keepdims=True)
    cent = h - mean
    var = jnp.mean(cent * cent, axis=0, keepdims=True)
    o_ref[...] = cent * lax.rsqrt(var + EPS_BN) * g_ref[...] + be_ref[...]


def _finalize(x, parts_a, parts_b, w1t, b12, w2t, b22, g2, be2):
    return pl.pallas_call(
        _finalize_body,
        out_shape=jax.ShapeDtypeStruct((N, D), jnp.float32),
    )(x, parts_a, parts_b, w1t, b12, w2t, b22, g2, be2)


def kernel(x, edge_index, edge_attr, Wl, bl, W1, b1, W2, b2, gamma, beta):
    pad_e = IDX_ROWS_H * ROW - E_HALF
    ar = jnp.arange(pad_e, dtype=jnp.int32)
    src_pad = ar % N
    dst_pad = N + ar % (N_PAD - N)
    src = edge_index[0]
    dst = edge_index[1]
    idx = []
    for h in range(2):
        sl = slice(h * E_HALF, (h + 1) * E_HALF)
        idx.append((
            jnp.concatenate([src[sl], src_pad]).reshape(IDX_ROWS_H, ROW),
            jnp.concatenate([dst[sl], dst_pad]).reshape(IDX_ROWS_H, ROW),
        ))
    wlt = Wl.T
    bl2 = bl.reshape(1, D)
    zeros = jnp.zeros((SLAB, D), jnp.float32)
    nblk = E_HALF // 2000
    p0 = _edge_linear(edge_attr, wlt, bl2, 0)
    p1 = _edge_linear(edge_attr, wlt, bl2, nblk)
    parts_a = _sc_aggregate(p0, x, idx[0][0], idx[0][1], zeros)
    parts_b = _sc_aggregate(p1, x, idx[1][0], idx[1][1], zeros)
    return _finalize(x, parts_a, parts_b, W1.T, b1.reshape(1, D), W2.T,
                     b2.reshape(1, D), gamma.reshape(1, D), beta.reshape(1, D))
